# Initial kernel scaffold; baseline (speedup 1.0000x reference)
import jax
import jax.numpy as jnp
from jax.experimental import pallas as pl


def kernel(p2, p3, p4, p5, p6, image_sizes, annotations, W_inter, b_inter, W_logit, b_logit, W_reg, b_reg):
    raise NotImplementedError("write your pallas kernel here")


# speedup vs baseline: 1.4615x; 1.4615x over previous
import jax
import jax.numpy as jnp
from jax.experimental import pallas as pl

_LEVELS = ((128, 8), (64, 16), (32, 32), (16, 16), (8, 8))
_TOTAL_PX = sum(h * h for h, _ in _LEVELS)


def _rpn_body(x2, x3, x4, x5, x6, wt, wh, bi, bh, out):
    xs_refs = (x2, x3, x4, x5, x6)
    bi_v = bi[0, :][None, :]
    bh_v = bh[0, :][None, :]
    off = 0
    for x_ref, (H, tr) in zip(xs_refs, _LEVELS):
        W = H
        R = H // tr
        for r in range(R):
            acc = None
            for kx in range(3):
                xw = x_ref[0, r * tr : r * tr + tr + 2, kx : kx + W, :]
                for ky in range(3):
                    xsl = jax.lax.slice_in_dim(xw, ky, ky + tr, axis=0)
                    xsl = xsl.reshape(tr * W, 256)
                    p = jnp.dot(xsl, wt[ky * 3 + kx],
                                preferred_element_type=jnp.float32)
                    acc = p if acc is None else acc + p
            inter = jnp.maximum(acc + bi_v, 0.0).astype(jnp.bfloat16)
            head = jnp.dot(inter, wh[...],
                           preferred_element_type=jnp.float32) + bh_v
            base = off + r * tr * W
            out[0, base : base + tr * W, :] = head
        off += H * W


def kernel(p2, p3, p4, p5, p6, image_sizes, annotations,
           W_inter, b_inter, W_logit, b_logit, W_reg, b_reg):
    del image_sizes, annotations
    feats = (p2, p3, p4, p5, p6)
    xpads = tuple(
        jnp.pad(jnp.transpose(x.astype(jnp.bfloat16), (0, 2, 3, 1)),
                ((0, 0), (1, 1), (1, 1), (0, 0)))
        for x in feats)
    wt = jnp.transpose(W_inter, (2, 3, 1, 0)).reshape(9, 256, 256)
    wt = wt.astype(jnp.bfloat16)
    wh = jnp.concatenate([W_logit[:, :, 0, 0].T, W_reg[:, :, 0, 0].T], axis=1)
    wh = wh.astype(jnp.bfloat16)
    bi = b_inter.reshape(1, 256).astype(jnp.float32)
    bh = jnp.concatenate([b_logit, b_reg]).reshape(1, 15).astype(jnp.float32)

    n = p2.shape[0]
    in_specs = [
        pl.BlockSpec((1,) + xp.shape[1:], lambda nn: (nn, 0, 0, 0))
        for xp in xpads
    ]
    in_specs += [
        pl.BlockSpec((9, 256, 256), lambda nn: (0, 0, 0)),
        pl.BlockSpec((256, 15), lambda nn: (0, 0)),
        pl.BlockSpec((1, 256), lambda nn: (0, 0)),
        pl.BlockSpec((1, 15), lambda nn: (0, 0)),
    ]
    out = pl.pallas_call(
        _rpn_body,
        grid=(n,),
        in_specs=in_specs,
        out_specs=pl.BlockSpec((1, _TOTAL_PX, 15), lambda nn: (nn, 0, 0)),
        out_shape=jax.ShapeDtypeStruct((n, _TOTAL_PX, 15), jnp.float32),
    )(*xpads, wt, wh, bi, bh)

    logits = out[:, :, :3].reshape(n, _TOTAL_PX * 3)
    deltas = out[:, :, 3:].reshape(n, _TOTAL_PX * 3, 4)
    return (logits, deltas)
